# baseline (speedup 1.0000x reference)
import functools

import jax
import jax.numpy as jnp
from jax import lax
from jax.experimental import pallas as pl
from jax.experimental.pallas import tpu as pltpu
from jax.experimental.pallas import tpu_sc as plsc

N_NODES = 10000
FEAT = 128
EMB = 64
BATCH = 1024
N_EDGES = 320000

NC = 2
NS = 16
NW = NC * NS

CHUNK = 128
CH_PER_W = 80
EPW = CH_PER_W * CHUNK
EPAD = EPW * NW
DEG_N = 10240
DPS = DEG_N // NS

ACC_ROWS = 10112
RPS = ACC_ROWS // NS

_THETAS = ((1.0, -1.0, 0.25), (0.0, 1.0, -0.5), (0.0, 0.0, 0.25))

_mesh = plsc.VectorSubcoreMesh(core_axis_name="c", subcore_axis_name="s")


@functools.partial(
    pl.kernel,
    out_type=jax.ShapeDtypeStruct((NC, ACC_ROWS, EMB), jnp.float32),
    mesh=_mesh,
    scratch_types=[
        pltpu.VMEM((CH_PER_W, CHUNK), jnp.int32),
        pltpu.VMEM((CH_PER_W, CHUNK), jnp.int32),
        pltpu.VMEM((CHUNK, EMB), jnp.float32),
        pltpu.VMEM((CHUNK, EMB), jnp.float32),
        pltpu.VMEM_SHARED((ACC_ROWS, EMB), jnp.float32),
        pltpu.VMEM_SHARED((ACC_ROWS, EMB), jnp.float32),
        pltpu.SemaphoreType.DMA,
        pltpu.SemaphoreType.DMA,
    ],
    compiler_params=pltpu.CompilerParams(use_tc_tiling_on_sc=False),
)
def _segsum64(xs_hbm, src_hbm, dst_hbm, out_hbm,
              sidx_v, didx_v, rows_v, rows_b, xs_sh, acc_sh, sem0, sem1):
    cid = lax.axis_index("c")
    sid = lax.axis_index("s")
    g = cid * NS + sid
    pltpu.sync_copy(src_hbm.at[g], sidx_v)
    pltpu.sync_copy(dst_hbm.at[g], didx_v)
    pltpu.sync_copy(xs_hbm.at[pl.ds(sid * RPS, RPS)],
                    xs_sh.at[pl.ds(sid * RPS, RPS)])
    def zbody(r, carry):
        for c in range(EMB // 16):
            rows_v[r, pl.ds(c * 16, 16)] = jnp.zeros((16,), jnp.float32)
        return carry

    lax.fori_loop(0, CHUNK, zbody, 0)
    for k in range(RPS // CHUNK):
        pltpu.sync_copy(rows_v, acc_sh.at[pl.ds(sid * RPS + k * CHUNK, CHUNK)])
    tail = RPS % CHUNK
    if tail:
        pltpu.sync_copy(
            rows_v.at[pl.ds(0, tail)],
            acc_sh.at[pl.ds(sid * RPS + (RPS // CHUNK) * CHUNK, tail)])
    plsc.subcore_barrier()

    last = CH_PER_W - 1
    pltpu.async_copy(xs_hbm.at[sidx_v.at[0]], rows_v, sem0)
    pltpu.async_copy(xs_sh.at[sidx_v.at[1]], rows_b, sem1)

    def body(i, carry):
        j0 = 2 * i
        j1 = 2 * i + 1
        jn0 = jnp.minimum(j0 + 2, last)
        jn1 = jnp.minimum(j1 + 2, last)
        pltpu.make_async_copy(xs_hbm.at[sidx_v.at[j0]], rows_v, sem0).wait()
        pltpu.sync_copy(rows_v, acc_sh.at[didx_v.at[j0]], add=True)
        pltpu.async_copy(xs_hbm.at[sidx_v.at[jn0]], rows_v, sem0)
        pltpu.make_async_copy(xs_sh.at[sidx_v.at[j1]], rows_b, sem1).wait()
        pltpu.sync_copy(rows_b, acc_sh.at[didx_v.at[j1]], add=True)
        pltpu.async_copy(xs_sh.at[sidx_v.at[jn1]], rows_b, sem1)
        return carry

    lax.fori_loop(0, CH_PER_W // 2, body, 0)
    pltpu.make_async_copy(xs_hbm.at[sidx_v.at[last]], rows_v, sem0).wait()
    pltpu.make_async_copy(xs_sh.at[sidx_v.at[last]], rows_b, sem1).wait()
    plsc.subcore_barrier()
    pltpu.sync_copy(acc_sh.at[pl.ds(sid * RPS, RPS)],
                    out_hbm.at[cid, pl.ds(sid * RPS, RPS)])


@functools.partial(
    pl.kernel,
    out_type=jax.ShapeDtypeStruct((NC, DEG_N), jnp.float32),
    mesh=_mesh,
    scratch_types=[
        pltpu.VMEM((CH_PER_W, CHUNK), jnp.int32),
        pltpu.VMEM((CHUNK,), jnp.float32),
        pltpu.VMEM((DPS,), jnp.float32),
        pltpu.VMEM_SHARED((DEG_N,), jnp.float32),
    ],
    compiler_params=pltpu.CompilerParams(use_tc_tiling_on_sc=False),
)
def _degcount(dst_hbm, zeros_hbm, out_hbm, didx_v, ones_v, stage_v, acc_sh):
    cid = lax.axis_index("c")
    sid = lax.axis_index("s")
    g = cid * NS + sid
    pltpu.sync_copy(dst_hbm.at[g], didx_v)
    for i in range(CHUNK // 16):
        ones_v[pl.ds(i * 16, 16)] = jnp.ones((16,), jnp.float32)
    pltpu.sync_copy(zeros_hbm.at[pl.ds(sid * DPS, DPS)], stage_v)
    pltpu.sync_copy(stage_v, acc_sh.at[pl.ds(sid * DPS, DPS)])
    plsc.subcore_barrier()

    def body(j, carry):
        pltpu.sync_copy(ones_v, acc_sh.at[didx_v.at[j]], add=True)
        return carry

    lax.fori_loop(0, CH_PER_W, body, 0)
    plsc.subcore_barrier()
    pltpu.sync_copy(acc_sh.at[pl.ds(sid * DPS, DPS)], stage_v)
    pltpu.sync_copy(stage_v, out_hbm.at[cid, pl.ds(sid * DPS, DPS)])


def _make_gather(D):
    bpw = BATCH // NW

    @functools.partial(
        pl.kernel,
        out_type=jax.ShapeDtypeStruct((BATCH, D), jnp.float32),
        mesh=_mesh,
        scratch_types=[
            pltpu.VMEM((bpw,), jnp.int32),
            pltpu.VMEM((bpw, D), jnp.float32),
            pltpu.SemaphoreType.DMA,
        ],
        compiler_params=pltpu.CompilerParams(use_tc_tiling_on_sc=False),
    )
    def _gather(table_hbm, idx_hbm, out_hbm, idx_v, rows_v, sem):
        cid = lax.axis_index("c")
        sid = lax.axis_index("s")
        base = (cid * NS + sid) * bpw
        pltpu.sync_copy(idx_hbm.at[pl.ds(base, bpw)], idx_v)
        pltpu.async_copy(table_hbm.at[idx_v], rows_v, sem).wait()
        pltpu.sync_copy(rows_v, out_hbm.at[pl.ds(base, bpw)])

    return _gather


_gather_feat = _make_gather(FEAT)
_gather_emb = _make_gather(EMB)


def _mlp_body(f_ref, w1_ref, b1_ref, w2_ref, b2_ref, deg2_ref,
              h_ref, y_ref, d_ref):
    f = f_ref[...]
    h = jnp.maximum(jnp.dot(f, w1_ref[...], preferred_element_type=jnp.float32)
                    + b1_ref[...][None, :], 0.0)
    h = jnp.maximum(jnp.dot(h, w2_ref[...], preferred_element_type=jnp.float32)
                    + b2_ref[...][None, :], 0.0)
    deg = deg2_ref[0, :N_NODES] + deg2_ref[1, :N_NODES]
    d = lax.rsqrt(jnp.maximum(deg, 1.0))
    h_ref[...] = h
    y_ref[:N_NODES, :] = d[:, None] * h
    y_ref[N_NODES:, :] = jnp.zeros((ACC_ROWS - N_NODES, EMB), jnp.float32)
    d_ref[...] = d[:, None]


def _mlp(features, W1, b1, W2, b2, deg2):
    return pl.pallas_call(
        _mlp_body,
        out_shape=(
            jax.ShapeDtypeStruct((N_NODES, EMB), jnp.float32),
            jax.ShapeDtypeStruct((ACC_ROWS, EMB), jnp.float32),
            jax.ShapeDtypeStruct((N_NODES, 1), jnp.float32),
        ),
    )(features, W1, b1, W2, b2, deg2)


def _combine_body(a, b, c, h_ref, t_ref, p2_ref, d_ref, out_ref, y_ref):
    p = p2_ref[0, :N_NODES, :] + p2_ref[1, :N_NODES, :]
    d = d_ref[...]
    t = t_ref[...]
    out = c * (t - d * p)
    if b != 0.0:
        out = out + b * t
    if a != 0.0:
        out = out + a * h_ref[...]
    out_ref[...] = out
    y_ref[:N_NODES, :] = d * out
    y_ref[N_NODES:, :] = jnp.zeros((ACC_ROWS - N_NODES, EMB), jnp.float32)


def _combine(a, b, c, h, t, p2, d):
    body = functools.partial(_combine_body, a, b, c)
    return pl.pallas_call(
        body,
        out_shape=(
            jax.ShapeDtypeStruct((N_NODES, EMB), jnp.float32),
            jax.ShapeDtypeStruct((ACC_ROWS, EMB), jnp.float32),
        ),
    )(h, t, p2, d)


def _final_body(fsel_ref, hsel_ref, w3_ref, b3_ref, w_ref, wclf_ref, bclf_ref,
                comb_ref, cs_ref):
    fsel = fsel_ref[...]
    spe = jnp.dot(hsel_ref[...], w3_ref[...],
                  preferred_element_type=jnp.float32) + b3_ref[...][None, :]
    center_h = jnp.dot(fsel, w_ref[...], preferred_element_type=jnp.float32)
    agg = jnp.dot(spe, w_ref[...], preferred_element_type=jnp.float32)
    comb_ref[...] = jnp.maximum(center_h + agg, 0.0)
    cs_ref[...] = jnp.dot(fsel, wclf_ref[...],
                          preferred_element_type=jnp.float32) + bclf_ref[...][None, :]


def _final(fsel, hsel, W3, b3, weight, W_clf, b_clf):
    return pl.pallas_call(
        _final_body,
        out_shape=(
            jax.ShapeDtypeStruct((BATCH, EMB), jnp.float32),
            jax.ShapeDtypeStruct((BATCH, 2), jnp.float32),
        ),
    )(fsel, hsel, W3, b3, weight, W_clf, b_clf)


def kernel(nodes, labels, edge_index, features, W_clf, b_clf,
           W1, b1, W2, b2, W3, b3, weight):
    src = edge_index[0]
    dst = edge_index[1]
    pad = EPAD - N_EDGES
    src_p = jnp.concatenate(
        [src, jnp.zeros((pad,), jnp.int32)]).reshape(NW, CH_PER_W, CHUNK)
    dst_p = jnp.concatenate(
        [dst, jnp.full((pad,), N_NODES, jnp.int32)]).reshape(NW, CH_PER_W, CHUNK)
    zerosd = jnp.zeros((DEG_N,), jnp.float32)

    deg2 = _degcount(dst_p, zerosd)

    h, y, d = _mlp(features, W1, b1, W2, b2, deg2)

    pad_y = jnp.zeros((ACC_ROWS - N_NODES, EMB), jnp.float32)
    for t0, t1, t2 in _THETAS:
        p2 = _segsum64(y, src_p, dst_p)
        p = p2[0, :N_NODES, :] + p2[1, :N_NODES, :]
        tmp1 = h - d * p
        y = jnp.concatenate([d * tmp1, pad_y], axis=0)
        p2 = _segsum64(y, src_p, dst_p)
        p = p2[0, :N_NODES, :] + p2[1, :N_NODES, :]
        h = t0 * h + t1 * tmp1 + t2 * (tmp1 - d * p)
        y = jnp.concatenate([d * h, pad_y], axis=0)

    fsel = _gather_feat(features, nodes)
    hsel = _gather_emb(h, nodes)
    combined, center_scores = _final(fsel, hsel, W3, b3, weight, W_clf, b_clf)
    return (combined, center_scores)


# speedup vs baseline: 1.5979x; 1.5979x over previous
import functools

import jax
import jax.numpy as jnp
from jax import lax
from jax.experimental import pallas as pl
from jax.experimental.pallas import tpu as pltpu
from jax.experimental.pallas import tpu_sc as plsc

N_NODES = 10000
FEAT = 128
EMB = 64
BATCH = 1024
N_EDGES = 320000

NC = 2
NS = 16
NW = NC * NS

CHUNK = 128
CH_PER_W = 80
EPW = CH_PER_W * CHUNK
EPAD = EPW * NW
DEG_N = 10240
DPS = DEG_N // NS

ACC_ROWS = 10112
RPS = ACC_ROWS // NS

_THETAS = ((1.0, -1.0, 0.25), (0.0, 1.0, -0.5), (0.0, 0.0, 0.25))

_mesh = plsc.VectorSubcoreMesh(core_axis_name="c", subcore_axis_name="s")


@functools.partial(
    pl.kernel,
    out_type=jax.ShapeDtypeStruct((NC, ACC_ROWS, EMB), jnp.float32),
    mesh=_mesh,
    scratch_types=[
        pltpu.VMEM((CH_PER_W, CHUNK), jnp.int32),
        pltpu.VMEM((CH_PER_W, CHUNK), jnp.int32),
        pltpu.VMEM((CHUNK, EMB), jnp.float32),
        pltpu.VMEM((CHUNK, EMB), jnp.float32),
        pltpu.VMEM_SHARED((ACC_ROWS, EMB), jnp.float32),
        pltpu.VMEM_SHARED((ACC_ROWS, EMB), jnp.float32),
        pltpu.SemaphoreType.DMA,
        pltpu.SemaphoreType.DMA,
    ],
    compiler_params=pltpu.CompilerParams(use_tc_tiling_on_sc=False),
)
def _segsum64(xs_hbm, src_hbm, dst_hbm, out_hbm,
              sidx_v, didx_v, rows_v, rows_b, xs_sh, acc_sh, sem0, sem1):
    cid = lax.axis_index("c")
    sid = lax.axis_index("s")
    g = cid * NS + sid
    pltpu.sync_copy(src_hbm.at[g], sidx_v)
    pltpu.sync_copy(dst_hbm.at[g], didx_v)
    pltpu.sync_copy(xs_hbm.at[pl.ds(sid * RPS, RPS)],
                    xs_sh.at[pl.ds(sid * RPS, RPS)])
    def zbody(r, carry):
        for c in range(EMB // 16):
            rows_v[r, pl.ds(c * 16, 16)] = jnp.zeros((16,), jnp.float32)
        return carry

    lax.fori_loop(0, CHUNK, zbody, 0)
    for k in range(RPS // CHUNK):
        pltpu.sync_copy(rows_v, acc_sh.at[pl.ds(sid * RPS + k * CHUNK, CHUNK)])
    tail = RPS % CHUNK
    if tail:
        pltpu.sync_copy(
            rows_v.at[pl.ds(0, tail)],
            acc_sh.at[pl.ds(sid * RPS + (RPS // CHUNK) * CHUNK, tail)])
    plsc.subcore_barrier()

    last = CH_PER_W - 1
    pltpu.async_copy(xs_sh.at[sidx_v.at[0]], rows_v, sem0)
    pltpu.async_copy(xs_sh.at[sidx_v.at[1]], rows_b, sem1)

    def body(i, carry):
        j0 = 2 * i
        j1 = 2 * i + 1
        jn0 = jnp.minimum(j0 + 2, last)
        jn1 = jnp.minimum(j1 + 2, last)
        pltpu.make_async_copy(xs_sh.at[sidx_v.at[j0]], rows_v, sem0).wait()
        pltpu.sync_copy(rows_v, acc_sh.at[didx_v.at[j0]], add=True)
        pltpu.async_copy(xs_sh.at[sidx_v.at[jn0]], rows_v, sem0)
        pltpu.make_async_copy(xs_sh.at[sidx_v.at[j1]], rows_b, sem1).wait()
        pltpu.sync_copy(rows_b, acc_sh.at[didx_v.at[j1]], add=True)
        pltpu.async_copy(xs_sh.at[sidx_v.at[jn1]], rows_b, sem1)
        return carry

    lax.fori_loop(0, CH_PER_W // 2, body, 0)
    pltpu.make_async_copy(xs_sh.at[sidx_v.at[last]], rows_v, sem0).wait()
    pltpu.make_async_copy(xs_sh.at[sidx_v.at[last]], rows_b, sem1).wait()
    plsc.subcore_barrier()
    pltpu.sync_copy(acc_sh.at[pl.ds(sid * RPS, RPS)],
                    out_hbm.at[cid, pl.ds(sid * RPS, RPS)])


@functools.partial(
    pl.kernel,
    out_type=jax.ShapeDtypeStruct((NC, DEG_N), jnp.float32),
    mesh=_mesh,
    scratch_types=[
        pltpu.VMEM((CH_PER_W, CHUNK), jnp.int32),
        pltpu.VMEM((CHUNK,), jnp.float32),
        pltpu.VMEM((DPS,), jnp.float32),
        pltpu.VMEM_SHARED((DEG_N,), jnp.float32),
    ],
    compiler_params=pltpu.CompilerParams(use_tc_tiling_on_sc=False),
)
def _degcount(dst_hbm, zeros_hbm, out_hbm, didx_v, ones_v, stage_v, acc_sh):
    cid = lax.axis_index("c")
    sid = lax.axis_index("s")
    g = cid * NS + sid
    pltpu.sync_copy(dst_hbm.at[g], didx_v)
    for i in range(CHUNK // 16):
        ones_v[pl.ds(i * 16, 16)] = jnp.ones((16,), jnp.float32)
    pltpu.sync_copy(zeros_hbm.at[pl.ds(sid * DPS, DPS)], stage_v)
    pltpu.sync_copy(stage_v, acc_sh.at[pl.ds(sid * DPS, DPS)])
    plsc.subcore_barrier()

    def body(j, carry):
        pltpu.sync_copy(ones_v, acc_sh.at[didx_v.at[j]], add=True)
        return carry

    lax.fori_loop(0, CH_PER_W, body, 0)
    plsc.subcore_barrier()
    pltpu.sync_copy(acc_sh.at[pl.ds(sid * DPS, DPS)], stage_v)
    pltpu.sync_copy(stage_v, out_hbm.at[cid, pl.ds(sid * DPS, DPS)])


def _make_gather(D):
    bpw = BATCH // NW

    @functools.partial(
        pl.kernel,
        out_type=jax.ShapeDtypeStruct((BATCH, D), jnp.float32),
        mesh=_mesh,
        scratch_types=[
            pltpu.VMEM((bpw,), jnp.int32),
            pltpu.VMEM((bpw, D), jnp.float32),
            pltpu.SemaphoreType.DMA,
        ],
        compiler_params=pltpu.CompilerParams(use_tc_tiling_on_sc=False),
    )
    def _gather(table_hbm, idx_hbm, out_hbm, idx_v, rows_v, sem):
        cid = lax.axis_index("c")
        sid = lax.axis_index("s")
        base = (cid * NS + sid) * bpw
        pltpu.sync_copy(idx_hbm.at[pl.ds(base, bpw)], idx_v)
        pltpu.async_copy(table_hbm.at[idx_v], rows_v, sem).wait()
        pltpu.sync_copy(rows_v, out_hbm.at[pl.ds(base, bpw)])

    return _gather


_gather_feat = _make_gather(FEAT)
_gather_emb = _make_gather(EMB)


def _mlp_body(f_ref, w1_ref, b1_ref, w2_ref, b2_ref, deg2_ref,
              h_ref, y_ref, d_ref):
    f = f_ref[...]
    h = jnp.maximum(jnp.dot(f, w1_ref[...], preferred_element_type=jnp.float32)
                    + b1_ref[...][None, :], 0.0)
    h = jnp.maximum(jnp.dot(h, w2_ref[...], preferred_element_type=jnp.float32)
                    + b2_ref[...][None, :], 0.0)
    deg = deg2_ref[0, :N_NODES] + deg2_ref[1, :N_NODES]
    d = lax.rsqrt(jnp.maximum(deg, 1.0))
    h_ref[...] = h
    y_ref[:N_NODES, :] = d[:, None] * h
    y_ref[N_NODES:, :] = jnp.zeros((ACC_ROWS - N_NODES, EMB), jnp.float32)
    d_ref[...] = d[:, None]


def _mlp(features, W1, b1, W2, b2, deg2):
    return pl.pallas_call(
        _mlp_body,
        out_shape=(
            jax.ShapeDtypeStruct((N_NODES, EMB), jnp.float32),
            jax.ShapeDtypeStruct((ACC_ROWS, EMB), jnp.float32),
            jax.ShapeDtypeStruct((N_NODES, 1), jnp.float32),
        ),
    )(features, W1, b1, W2, b2, deg2)


def _combine_body(a, b, c, h_ref, t_ref, p2_ref, d_ref, out_ref, y_ref):
    p = p2_ref[0, :N_NODES, :] + p2_ref[1, :N_NODES, :]
    d = d_ref[...]
    t = t_ref[...]
    out = c * (t - d * p)
    if b != 0.0:
        out = out + b * t
    if a != 0.0:
        out = out + a * h_ref[...]
    out_ref[...] = out
    y_ref[:N_NODES, :] = d * out
    y_ref[N_NODES:, :] = jnp.zeros((ACC_ROWS - N_NODES, EMB), jnp.float32)


def _combine(a, b, c, h, t, p2, d):
    body = functools.partial(_combine_body, a, b, c)
    return pl.pallas_call(
        body,
        out_shape=(
            jax.ShapeDtypeStruct((N_NODES, EMB), jnp.float32),
            jax.ShapeDtypeStruct((ACC_ROWS, EMB), jnp.float32),
        ),
    )(h, t, p2, d)


def _final_body(fsel_ref, hsel_ref, w3_ref, b3_ref, w_ref, wclf_ref, bclf_ref,
                comb_ref, cs_ref):
    fsel = fsel_ref[...]
    spe = jnp.dot(hsel_ref[...], w3_ref[...],
                  preferred_element_type=jnp.float32) + b3_ref[...][None, :]
    center_h = jnp.dot(fsel, w_ref[...], preferred_element_type=jnp.float32)
    agg = jnp.dot(spe, w_ref[...], preferred_element_type=jnp.float32)
    comb_ref[...] = jnp.maximum(center_h + agg, 0.0)
    cs_ref[...] = jnp.dot(fsel, wclf_ref[...],
                          preferred_element_type=jnp.float32) + bclf_ref[...][None, :]


def _final(fsel, hsel, W3, b3, weight, W_clf, b_clf):
    return pl.pallas_call(
        _final_body,
        out_shape=(
            jax.ShapeDtypeStruct((BATCH, EMB), jnp.float32),
            jax.ShapeDtypeStruct((BATCH, 2), jnp.float32),
        ),
    )(fsel, hsel, W3, b3, weight, W_clf, b_clf)


def kernel(nodes, labels, edge_index, features, W_clf, b_clf,
           W1, b1, W2, b2, W3, b3, weight):
    src = edge_index[0]
    dst = edge_index[1]
    pad = EPAD - N_EDGES
    src_p = jnp.concatenate(
        [src, jnp.zeros((pad,), jnp.int32)]).reshape(NW, CH_PER_W, CHUNK)
    dst_p = jnp.concatenate(
        [dst, jnp.full((pad,), N_NODES, jnp.int32)]).reshape(NW, CH_PER_W, CHUNK)
    zerosd = jnp.zeros((DEG_N,), jnp.float32)

    deg2 = _degcount(dst_p, zerosd)

    h, y, d = _mlp(features, W1, b1, W2, b2, deg2)

    pad_y = jnp.zeros((ACC_ROWS - N_NODES, EMB), jnp.float32)
    for t0, t1, t2 in _THETAS:
        p2 = _segsum64(y, src_p, dst_p)
        p = p2[0, :N_NODES, :] + p2[1, :N_NODES, :]
        tmp1 = h - d * p
        y = jnp.concatenate([d * tmp1, pad_y], axis=0)
        p2 = _segsum64(y, src_p, dst_p)
        p = p2[0, :N_NODES, :] + p2[1, :N_NODES, :]
        h = t0 * h + t1 * tmp1 + t2 * (tmp1 - d * p)
        y = jnp.concatenate([d * h, pad_y], axis=0)

    fsel = _gather_feat(features, nodes)
    hsel = _gather_emb(h, nodes)
    combined, center_scores = _final(fsel, hsel, W3, b3, weight, W_clf, b_clf)
    return (combined, center_scores)
